# Initial kernel scaffold; baseline (speedup 1.0000x reference)
import jax
import jax.numpy as jnp
from jax.experimental import pallas as pl


def kernel(input):
    raise NotImplementedError("write your pallas kernel here")


# speedup vs baseline: 11.4221x; 11.4221x over previous
import jax
import jax.numpy as jnp
from jax.experimental import pallas as pl
from jax.experimental.pallas import tpu as pltpu

_IOU_THR = 0.5
_ROIS = 256
_SCORE_THR = -1e9
_LANES = 128


def _nms_body(s_ref, x_ref, y_ref, w_ref, h_ref, out_ref,
              x1_ref, y1_ref, x2_ref, y2_ref, a3_ref, msc_ref):
    rows = s_ref.shape[1]
    x = x_ref[0]
    y = y_ref[0]
    w = w_ref[0]
    h = h_ref[0]
    ws = jnp.floor(w * 0.5)
    hs = jnp.floor(h * 0.5)
    x1 = x - ws
    x2 = x + ws
    y1 = y - hs
    y2 = y + hs
    x1_ref[...] = x1
    y1_ref[...] = y1
    x2_ref[...] = x2
    y2_ref[...] = y2
    a3_ref[...] = (x2 - x1) * (y2 - y1) * (1.0 / 3.0)
    s = s_ref[0]
    msc_ref[...] = jnp.where(s > _SCORE_THR, s, -jnp.inf)

    iota = (jax.lax.broadcasted_iota(jnp.int32, (rows, _LANES), 0) * _LANES
            + jax.lax.broadcasted_iota(jnp.int32, (rows, _LANES), 1))
    lane = jax.lax.broadcasted_iota(jnp.int32, (1, _LANES), 1)
    big = jnp.int32(rows * _LANES)

    def step(k, _):
        msc = msc_ref[...]
        m = jnp.max(msc)
        idx = jnp.min(jnp.where(msc == m, iota, big))
        has = m > -jnp.inf
        row = idx // _LANES
        col = idx - row * _LANES
        pick = lane == col
        bx1 = jnp.sum(jnp.where(pick, x1_ref[pl.ds(row, 1), :], 0.0))
        by1 = jnp.sum(jnp.where(pick, y1_ref[pl.ds(row, 1), :], 0.0))
        bx2 = jnp.sum(jnp.where(pick, x2_ref[pl.ds(row, 1), :], 0.0))
        by2 = jnp.sum(jnp.where(pick, y2_ref[pl.ds(row, 1), :], 0.0))
        ba3 = jnp.sum(jnp.where(pick, a3_ref[pl.ds(row, 1), :], 0.0))
        ix1 = jnp.maximum(x1_ref[...], bx1)
        iy1 = jnp.maximum(y1_ref[...], by1)
        ix2 = jnp.minimum(x2_ref[...], bx2)
        iy2 = jnp.minimum(y2_ref[...], by2)
        inter = jnp.maximum(ix2 - ix1, 0.0) * jnp.maximum(iy2 - iy1, 0.0)
        supp = inter > (a3_ref[...] + ba3)
        kill = jnp.logical_and(jnp.logical_or(supp, iota == idx), has)
        msc_ref[...] = jnp.where(kill, -jnp.inf, msc)
        out_ref[0, 0, k] = jnp.where(has, idx, jnp.int32(-1))
        return 0

    jax.lax.fori_loop(0, _ROIS, step, 0)


def kernel(input):
    b, n, _ = input.shape
    rows = (n + _LANES - 1) // _LANES
    npad = rows * _LANES
    pad = npad - n

    s = jnp.pad(input[:, :, 0], ((0, 0), (0, pad)), constant_values=-jnp.inf)
    x = jnp.pad(input[:, :, 1], ((0, 0), (0, pad)))
    y = jnp.pad(input[:, :, 2], ((0, 0), (0, pad)))
    w = jnp.pad(input[:, :, 3], ((0, 0), (0, pad)))
    h = jnp.pad(input[:, :, 4], ((0, 0), (0, pad)))
    shape3 = (b, rows, _LANES)
    s, x, y, w, h = (a.reshape(shape3) for a in (s, x, y, w, h))

    spec = pl.BlockSpec((1, rows, _LANES), lambda i: (i, 0, 0))
    sels = pl.pallas_call(
        _nms_body,
        grid=(b,),
        in_specs=[spec] * 5,
        out_specs=pl.BlockSpec(
            (1, 1, _ROIS), lambda i: (i, 0, 0), memory_space=pltpu.SMEM),
        out_shape=jax.ShapeDtypeStruct((b, 1, _ROIS), jnp.int32),
        scratch_shapes=[pltpu.VMEM((rows, _LANES), jnp.float32)] * 6,
        compiler_params=pltpu.CompilerParams(
            dimension_semantics=("arbitrary",)),
    )(s, x, y, w, h)
    sels = sels.reshape(b, _ROIS)

    keys = jax.random.split(jax.random.key(1), b)
    rand = jax.vmap(
        lambda k: jax.random.randint(k, (_ROIS,), 0, n, dtype=jnp.int32))(keys)
    return jnp.where(sels >= 0, sels, rand)


# speedup vs baseline: 11.7755x; 1.0309x over previous
import jax
import jax.numpy as jnp
from jax.experimental import pallas as pl
from jax.experimental.pallas import tpu as pltpu

_IOU_THR = 0.5
_ROIS = 256
_SCORE_THR = -1e9
_LANES = 128


def _nms_body(s_ref, x_ref, y_ref, w_ref, h_ref, out_ref,
              x1_ref, y1_ref, x2_ref, y2_ref, a3_ref):
    rows = s_ref.shape[1]
    x = x_ref[0]
    y = y_ref[0]
    w = w_ref[0]
    h = h_ref[0]
    ws = jnp.floor(w * 0.5)
    hs = jnp.floor(h * 0.5)
    x1 = x - ws
    x2 = x + ws
    y1 = y - hs
    y2 = y + hs
    x1_ref[...] = x1
    y1_ref[...] = y1
    x2_ref[...] = x2
    y2_ref[...] = y2
    a3_ref[...] = (x2 - x1) * (y2 - y1) * (1.0 / 3.0)
    s = s_ref[0]
    msc0 = jnp.where(s > _SCORE_THR, s, -jnp.inf)

    iota = (jax.lax.broadcasted_iota(jnp.int32, (rows, _LANES), 0) * _LANES
            + jax.lax.broadcasted_iota(jnp.int32, (rows, _LANES), 1))
    lane = jax.lax.broadcasted_iota(jnp.int32, (1, _LANES), 1)
    big = jnp.int32(rows * _LANES)

    def step(k, msc):
        m = jnp.max(msc)
        idx = jnp.min(jnp.where(msc == m, iota, big))
        has = m > -jnp.inf
        row = idx // _LANES
        col = idx - row * _LANES
        pick = lane == col
        bx1 = jnp.sum(jnp.where(pick, x1_ref[pl.ds(row, 1), :], 0.0))
        by1 = jnp.sum(jnp.where(pick, y1_ref[pl.ds(row, 1), :], 0.0))
        bx2 = jnp.sum(jnp.where(pick, x2_ref[pl.ds(row, 1), :], 0.0))
        by2 = jnp.sum(jnp.where(pick, y2_ref[pl.ds(row, 1), :], 0.0))
        ba3 = jnp.sum(jnp.where(pick, a3_ref[pl.ds(row, 1), :], 0.0))
        ix1 = jnp.maximum(x1_ref[...], bx1)
        iy1 = jnp.maximum(y1_ref[...], by1)
        ix2 = jnp.minimum(x2_ref[...], bx2)
        iy2 = jnp.minimum(y2_ref[...], by2)
        inter = jnp.maximum(ix2 - ix1, 0.0) * jnp.maximum(iy2 - iy1, 0.0)
        kill = jnp.logical_and(inter > (a3_ref[...] + ba3), has)
        out_ref[0, 0, k] = jnp.where(has, idx, jnp.int32(-1))
        return jnp.where(kill, -jnp.inf, msc)

    jax.lax.fori_loop(0, _ROIS, step, msc0)


def kernel(input):
    b, n, _ = input.shape
    rows = (n + _LANES - 1) // _LANES
    npad = rows * _LANES
    pad = npad - n

    s = jnp.pad(input[:, :, 0], ((0, 0), (0, pad)), constant_values=-jnp.inf)
    x = jnp.pad(input[:, :, 1], ((0, 0), (0, pad)))
    y = jnp.pad(input[:, :, 2], ((0, 0), (0, pad)))
    w = jnp.pad(input[:, :, 3], ((0, 0), (0, pad)))
    h = jnp.pad(input[:, :, 4], ((0, 0), (0, pad)))
    shape3 = (b, rows, _LANES)
    s, x, y, w, h = (a.reshape(shape3) for a in (s, x, y, w, h))

    spec = pl.BlockSpec((1, rows, _LANES), lambda i: (i, 0, 0))
    sels = pl.pallas_call(
        _nms_body,
        grid=(b,),
        in_specs=[spec] * 5,
        out_specs=pl.BlockSpec(
            (1, 1, _ROIS), lambda i: (i, 0, 0), memory_space=pltpu.SMEM),
        out_shape=jax.ShapeDtypeStruct((b, 1, _ROIS), jnp.int32),
        scratch_shapes=[pltpu.VMEM((rows, _LANES), jnp.float32)] * 5,
        compiler_params=pltpu.CompilerParams(
            dimension_semantics=("parallel",)),
    )(s, x, y, w, h)
    sels = sels.reshape(b, _ROIS)

    keys = jax.random.split(jax.random.key(1), b)
    rand = jax.vmap(
        lambda k: jax.random.randint(k, (_ROIS,), 0, n, dtype=jnp.int32))(keys)
    return jnp.where(sels >= 0, sels, rand)
